# Initial kernel scaffold; baseline (speedup 1.0000x reference)
import jax
import jax.numpy as jnp
from jax.experimental import pallas as pl


def kernel(x, W):
    raise NotImplementedError("write your pallas kernel here")


# speedup vs baseline: 1.3494x; 1.3494x over previous
import functools

import jax
import jax.numpy as jnp
from jax.experimental import pallas as pl

INPUT_DIM = 2048
NUM_EXPERTS = 64
BLOCK_T = 1024


def _router_body(x_ref, wt_ref, disp_ref, probs_ref, sel_ref, w_ref):
    logits = jnp.dot(x_ref[...], wt_ref[...], preferred_element_type=jnp.float32)
    m = jnp.max(logits, axis=1, keepdims=True)
    e = jnp.exp(logits - m)
    probs = e / jnp.sum(e, axis=1, keepdims=True)
    probs_ref[...] = probs

    eid = jax.lax.broadcasted_iota(jnp.int32, probs.shape, 1)
    p1 = jnp.max(probs, axis=1, keepdims=True)
    i1 = jnp.min(jnp.where(probs == p1, eid, NUM_EXPERTS), axis=1, keepdims=True)
    masked = jnp.where(eid == i1, -1.0, probs)
    p2 = jnp.max(masked, axis=1, keepdims=True)
    i2 = jnp.min(jnp.where(masked == p2, eid, NUM_EXPERTS), axis=1, keepdims=True)

    denom = p1 + p2
    w1 = p1 / denom
    w2 = p2 / denom
    disp_ref[...] = jnp.where(
        eid == i1, w1, jnp.where(eid == i2, w2, jnp.zeros_like(probs))
    )
    sel_ref[...] = jnp.concatenate([i1, i2], axis=1)
    w_ref[...] = jnp.concatenate([w1, w2], axis=1)


@jax.jit
def kernel(x, W):
    B, S, D = x.shape
    T = B * S
    x2 = x.reshape(T, D)
    wt = W.T
    grid = (T // BLOCK_T,)
    disp, probs, sel, wts = pl.pallas_call(
        _router_body,
        grid=grid,
        in_specs=[
            pl.BlockSpec((BLOCK_T, D), lambda i: (i, 0)),
            pl.BlockSpec((D, NUM_EXPERTS), lambda i: (0, 0)),
        ],
        out_specs=[
            pl.BlockSpec((BLOCK_T, NUM_EXPERTS), lambda i: (i, 0)),
            pl.BlockSpec((BLOCK_T, NUM_EXPERTS), lambda i: (i, 0)),
            pl.BlockSpec((BLOCK_T, 2), lambda i: (i, 0)),
            pl.BlockSpec((BLOCK_T, 2), lambda i: (i, 0)),
        ],
        out_shape=[
            jax.ShapeDtypeStruct((T, NUM_EXPERTS), jnp.float32),
            jax.ShapeDtypeStruct((T, NUM_EXPERTS), jnp.float32),
            jax.ShapeDtypeStruct((T, 2), jnp.int32),
            jax.ShapeDtypeStruct((T, 2), jnp.float32),
        ],
    )(x2, wt)
    return (
        disp.reshape(B, S, NUM_EXPERTS),
        probs.reshape(B, S, NUM_EXPERTS),
        sel.reshape(B, S, 2),
        wts.reshape(B, S, 2),
    )
